# baseline (speedup 1.0000x reference)
import functools

import jax
import jax.numpy as jnp
from jax import lax
from jax.experimental import pallas as pl
from jax.experimental.pallas import tpu as pltpu
from jax.experimental.pallas import tpu_sc as plsc

B, S, V, H = 4, 2048, 128100, 1024
NT = B * S
LN_EPS = 1e-07
L = 16
CC = H // L

_info = plsc.get_sparse_core_info()
NC, NS = _info.num_cores, _info.num_subcores
NW = NC * NS
POS_W = S // NW
CHUNK = 16
SUBS = POS_W // CHUNK
N_CHUNKS = B * SUBS
NBUF = 2
RG = 8


def _hsum(v):
    ii = lax.iota(jnp.int32, L)
    dnums = lax.GatherDimensionNumbers(
        offset_dims=(), collapsed_slice_dims=(0,), start_index_map=(0,))
    for sh in (8, 4, 2, 1):
        perm = (ii ^ sh).reshape(L, 1)
        v = v + lax.gather(v, perm, dnums, (1,),
                           mode=lax.GatherScatterMode.PROMISE_IN_BOUNDS)
    return v


def _vrsqrt(x):
    i = lax.bitcast_convert_type(x, jnp.int32)
    i = jnp.int32(0x5F3759DF) - lax.shift_right_logical(i, 1)
    y = lax.bitcast_convert_type(i, jnp.float32)
    for _ in range(3):
        y = y * (1.5 - 0.5 * x * y * y)
    return y


def _sc_fused(idx_grouped, table, pos, scale, bias):
    mesh = plsc.VectorSubcoreMesh(core_axis_name="c", subcore_axis_name="s")

    @functools.partial(
        pl.kernel,
        mesh=mesh,
        out_type=jax.ShapeDtypeStruct((NT, H), jnp.float32),
        scratch_types=[
            pltpu.VMEM((N_CHUNKS, CHUNK), jnp.int32),
            pltpu.VMEM((NBUF, CHUNK, H), jnp.float32),
            pltpu.VMEM((POS_W, H), jnp.float32),
            pltpu.VMEM((H,), jnp.float32),
            pltpu.VMEM((H,), jnp.float32),
            pltpu.VMEM((CHUNK, L), jnp.float32),
            pltpu.VMEM((CHUNK, L), jnp.float32),
            pltpu.SemaphoreType.DMA,
            pltpu.SemaphoreType.DMA,
            pltpu.SemaphoreType.DMA,
            pltpu.SemaphoreType.DMA,
        ],
    )
    def k(idx_hbm, table_hbm, pos_hbm, scale_hbm, bias_hbm, out_hbm,
          idx_v, rows_v, pos_v, scale_v, bias_v, a_v, b_v,
          g0, g1, w0, w1):
        wid = lax.axis_index("s") * NC + lax.axis_index("c")
        gsem = (g0, g1)
        wsem = (w0, w1)
        pltpu.sync_copy(idx_hbm.at[wid], idx_v)
        pltpu.sync_copy(pos_hbm.at[pl.ds(wid * POS_W, POS_W)], pos_v)
        pltpu.sync_copy(scale_hbm, scale_v)
        pltpu.sync_copy(bias_hbm, bias_v)

        def out_off(c):
            b, sub = divmod(c, SUBS)
            return b * S + wid * POS_W + sub * CHUNK

        gh = [None] * N_CHUNKS
        wh = [None] * N_CHUNKS
        gh[0] = pltpu.async_copy(
            table_hbm.at[idx_v.at[0]], rows_v.at[0], gsem[0])

        for c in range(N_CHUNKS):
            buf = c % NBUF
            nxt = (c + 1) % NBUF
            if c + 1 < N_CHUNKS:
                if c >= 1:
                    wh[c - 1].wait()
                gh[c + 1] = pltpu.async_copy(
                    table_hbm.at[idx_v.at[c + 1]], rows_v.at[nxt], gsem[nxt])
            gh[c].wait()

            x = rows_v.at[buf]
            sub = c % SUBS
            pbase = sub * CHUNK

            def p1_row(r, _):
                def p1_cc(kk, carry):
                    s_acc, q_acc = carry
                    col = pl.ds(kk * L, L)
                    v = x[r, col] + pos_v[pbase + r, col]
                    x[r, col] = v
                    return s_acc + v, q_acc + v * v
                s_acc, q_acc = lax.fori_loop(
                    0, CC, p1_cc,
                    (jnp.zeros((L,), jnp.float32), jnp.zeros((L,), jnp.float32)),
                    unroll=4)
                s_tot = _hsum(s_acc)
                q_tot = _hsum(q_acc)
                mean = s_tot * (1.0 / H)
                var = q_tot * (1.0 / H) - mean * mean
                rstd = _vrsqrt(var + LN_EPS)
                a_v[r, :] = rstd
                b_v[r, :] = -mean * rstd
                return 0
            lax.fori_loop(0, CHUNK, p1_row, 0)

            for r0 in range(0, CHUNK, RG):
                ars = [a_v[r0 + r, :] for r in range(RG)]
                brs = [b_v[r0 + r, :] for r in range(RG)]

                def p2_cc(kk, _, ars=ars, brs=brs, r0=r0):
                    col = pl.ds(kk * L, L)
                    sc = scale_v[col]
                    bi = bias_v[col]
                    for r in range(RG):
                        v = x[r0 + r, col] * ars[r] + brs[r]
                        x[r0 + r, col] = v * sc + bi
                    return 0
                lax.fori_loop(0, CC, p2_cc, 0)

            wh[c] = pltpu.async_copy(
                x, out_hbm.at[pl.ds(out_off(c), CHUNK)], wsem[buf])

        wh[N_CHUNKS - 2].wait()
        wh[N_CHUNKS - 1].wait()

    return k(idx_grouped, table, pos, scale, bias)


def kernel(input_ids, word_embeddings, position_embeddings, ln_scale, ln_bias):
    ids = input_ids.astype(jnp.int32).reshape(B, NW, SUBS, CHUNK)
    idx_grouped = ids.transpose(1, 0, 2, 3).reshape(NW, N_CHUNKS, CHUNK)
    out = _sc_fused(idx_grouped, word_embeddings, position_embeddings,
                    ln_scale, ln_bias)
    return out.reshape(B, S, H)


# speedup vs baseline: 1.8371x; 1.8371x over previous
import functools

import jax
import jax.numpy as jnp
from jax import lax
from jax.experimental import pallas as pl
from jax.experimental.pallas import tpu as pltpu
from jax.experimental.pallas import tpu_sc as plsc

B, S, V, H = 4, 2048, 128100, 1024
NT = B * S
LN_EPS = 1e-07

_info = plsc.get_sparse_core_info()
NC, NS = _info.num_cores, _info.num_subcores
NW = NC * NS
T_PER_W = S // NW
CHUNK = 32
N_CHUNKS = T_PER_W // CHUNK


def _sc_gather_slice(idx_grouped, table):
    mesh = plsc.VectorSubcoreMesh(core_axis_name="c", subcore_axis_name="s")

    @functools.partial(
        pl.kernel,
        mesh=mesh,
        out_type=jax.ShapeDtypeStruct((S, H), jnp.float32),
        scratch_types=[
            pltpu.VMEM((N_CHUNKS, CHUNK), jnp.int32),
            pltpu.VMEM((N_CHUNKS, CHUNK, H), jnp.float32),
            pltpu.SemaphoreType.DMA,
            pltpu.SemaphoreType.DMA,
            pltpu.SemaphoreType.DMA,
            pltpu.SemaphoreType.DMA,
        ],
    )
    def k(idx_hbm, table_hbm, out_hbm, idx_v, rows_v, g0, g1, w0, w1):
        wid = lax.axis_index("s") * NC + lax.axis_index("c")
        base = wid * T_PER_W
        gsem = (g0, g1)
        wsem = (w0, w1)
        pltpu.sync_copy(idx_hbm.at[wid], idx_v)
        gh = [pltpu.async_copy(table_hbm.at[idx_v.at[c]], rows_v.at[c], gsem[c])
              for c in range(N_CHUNKS)]
        wh = []
        for c in range(N_CHUNKS):
            gh[c].wait()
            wh.append(pltpu.async_copy(
                rows_v.at[c],
                out_hbm.at[pl.ds(base + c * CHUNK, CHUNK)],
                wsem[c]))
        for h in wh:
            h.wait()

    return k(idx_grouped, table)


ROWS_BLK = 256
N_BLK = S // ROWS_BLK


def _tc_add_ln_slice(gathered, pos, scale, bias, buf, slice_idx):
    def body(g_ref, p_ref, s_ref, b_ref, _buf_ref, o_ref):
        x = g_ref[...] + p_ref[...]
        mean = jnp.mean(x, axis=-1, keepdims=True)
        var = jnp.mean(jnp.square(x - mean), axis=-1, keepdims=True)
        normed = (x - mean) * lax.rsqrt(var + LN_EPS)
        o_ref[...] = normed * s_ref[...] + b_ref[...]

    return pl.pallas_call(
        body,
        grid=(N_BLK,),
        in_specs=[
            pl.BlockSpec((ROWS_BLK, H), lambda i: (i, 0)),
            pl.BlockSpec((ROWS_BLK, H), lambda i: (i, 0)),
            pl.BlockSpec((1, H), lambda i: (0, 0)),
            pl.BlockSpec((1, H), lambda i: (0, 0)),
            pl.BlockSpec(memory_space=pl.ANY),
        ],
        out_specs=pl.BlockSpec(
            (ROWS_BLK, H), lambda i, s=slice_idx: (s * N_BLK + i, 0)),
        out_shape=jax.ShapeDtypeStruct((NT, H), jnp.float32),
        input_output_aliases={4: 0},
    )(gathered, pos, scale, bias, buf)


def _tc_add_ln_first(gathered, pos, scale, bias):
    def body(g_ref, p_ref, s_ref, b_ref, o_ref):
        x = g_ref[...] + p_ref[...]
        mean = jnp.mean(x, axis=-1, keepdims=True)
        var = jnp.mean(jnp.square(x - mean), axis=-1, keepdims=True)
        normed = (x - mean) * lax.rsqrt(var + LN_EPS)
        o_ref[...] = normed * s_ref[...] + b_ref[...]

    return pl.pallas_call(
        body,
        grid=(N_BLK,),
        in_specs=[
            pl.BlockSpec((ROWS_BLK, H), lambda i: (i, 0)),
            pl.BlockSpec((ROWS_BLK, H), lambda i: (i, 0)),
            pl.BlockSpec((1, H), lambda i: (0, 0)),
            pl.BlockSpec((1, H), lambda i: (0, 0)),
        ],
        out_specs=pl.BlockSpec((ROWS_BLK, H), lambda i: (i, 0)),
        out_shape=jax.ShapeDtypeStruct((NT, H), jnp.float32),
    )(gathered, pos, scale, bias)


def kernel(input_ids, word_embeddings, position_embeddings, ln_scale, ln_bias):
    ids = input_ids.astype(jnp.int32).reshape(B, NW, N_CHUNKS, CHUNK)
    scale2 = ln_scale.reshape(1, H)
    bias2 = ln_bias.reshape(1, H)
    gathered = [_sc_gather_slice(ids[b], word_embeddings) for b in range(B)]
    buf = _tc_add_ln_first(gathered[0], position_embeddings, scale2, bias2)
    for b in range(1, B):
        buf = _tc_add_ln_slice(gathered[b], position_embeddings,
                               scale2, bias2, buf, b)
    return buf.reshape(B, S, H)


# speedup vs baseline: 1.9624x; 1.0682x over previous
import functools

import jax
import jax.numpy as jnp
from jax import lax
from jax.experimental import pallas as pl
from jax.experimental.pallas import tpu as pltpu
from jax.experimental.pallas import tpu_sc as plsc

B, S, V, H = 4, 2048, 128100, 1024
NT = B * S
LN_EPS = 1e-07

_info = plsc.get_sparse_core_info()
NC, NS = _info.num_cores, _info.num_subcores
NW = NC * NS
NSLICE = 4
QS = S // NSLICE
TS = B * QS
WPB = NW // B
T_PER_W = TS // NW
CHUNK = 32
N_CHUNKS = T_PER_W // CHUNK


def _sc_gather_slice(idx_grouped, table):
    mesh = plsc.VectorSubcoreMesh(core_axis_name="c", subcore_axis_name="s")

    @functools.partial(
        pl.kernel,
        mesh=mesh,
        out_type=jax.ShapeDtypeStruct((TS, H), jnp.float32),
        scratch_types=[
            pltpu.VMEM((N_CHUNKS, CHUNK), jnp.int32),
            pltpu.VMEM((N_CHUNKS, CHUNK, H), jnp.float32),
            pltpu.SemaphoreType.DMA,
            pltpu.SemaphoreType.DMA,
            pltpu.SemaphoreType.DMA,
            pltpu.SemaphoreType.DMA,
        ],
    )
    def k(idx_hbm, table_hbm, out_hbm, idx_v, rows_v, g0, g1, w0, w1):
        wid = lax.axis_index("s") * NC + lax.axis_index("c")
        base = wid * T_PER_W
        gsem = (g0, g1)
        wsem = (w0, w1)
        pltpu.sync_copy(idx_hbm.at[wid], idx_v)
        gh = [pltpu.async_copy(table_hbm.at[idx_v.at[c]], rows_v.at[c], gsem[c])
              for c in range(N_CHUNKS)]
        wh = []
        for c in range(N_CHUNKS):
            gh[c].wait()
            wh.append(pltpu.async_copy(
                rows_v.at[c],
                out_hbm.at[pl.ds(base + c * CHUNK, CHUNK)],
                wsem[c]))
        for h in wh:
            h.wait()

    return k(idx_grouped, table)


ROWS_BLK = 256
PB = QS // ROWS_BLK


def _ln_body(g_ref, p_ref, s_ref, b_ref, *rest):
    o_ref = rest[-1]
    x = g_ref[...] + p_ref[...]
    mean = jnp.mean(x, axis=-1, keepdims=True)
    var = jnp.mean(jnp.square(x - mean), axis=-1, keepdims=True)
    normed = (x - mean) * lax.rsqrt(var + LN_EPS)
    o_ref[...] = normed * s_ref[...] + b_ref[...]


def _tc_add_ln_slice(gathered, pos, scale, bias, buf, s):
    operands = [gathered, pos, scale, bias]
    in_specs = [
        pl.BlockSpec((ROWS_BLK, H), lambda i, j: (j * PB + i, 0)),
        pl.BlockSpec((ROWS_BLK, H), lambda i, j, s=s: (s * PB + i, 0)),
        pl.BlockSpec((1, H), lambda i, j: (0, 0)),
        pl.BlockSpec((1, H), lambda i, j: (0, 0)),
    ]
    aliases = {}
    if buf is not None:
        operands.append(buf)
        in_specs.append(pl.BlockSpec(memory_space=pl.ANY))
        aliases = {4: 0}
    return pl.pallas_call(
        _ln_body,
        grid=(PB, B),
        in_specs=in_specs,
        out_specs=pl.BlockSpec(
            (ROWS_BLK, H),
            lambda i, j, s=s: (j * (S // ROWS_BLK) + s * PB + i, 0)),
        out_shape=jax.ShapeDtypeStruct((NT, H), jnp.float32),
        input_output_aliases=aliases,
    )(*operands)


def kernel(input_ids, word_embeddings, position_embeddings, ln_scale, ln_bias):
    ids5 = input_ids.astype(jnp.int32).reshape(B, NSLICE, WPB, N_CHUNKS, CHUNK)
    scale2 = ln_scale.reshape(1, H)
    bias2 = ln_bias.reshape(1, H)
    gathered = [
        _sc_gather_slice(ids5[:, s].reshape(NW, N_CHUNKS, CHUNK),
                         word_embeddings)
        for s in range(NSLICE)
    ]
    buf = None
    for s in range(NSLICE):
        buf = _tc_add_ln_slice(gathered[s], position_embeddings,
                               scale2, bias2, buf, s)
    return buf.reshape(B, S, H)
